# Initial kernel scaffold; baseline (speedup 1.0000x reference)
import jax
import jax.numpy as jnp
from jax.experimental import pallas as pl


def kernel(x, edge_index, W_self1, W_neigh1, b1, W_self2, W_neigh2, b2, Wc1, bc1, Wc2, bc2):
    raise NotImplementedError("write your pallas kernel here")


# speedup vs baseline: 4.0643x; 4.0643x over previous
import jax
import jax.numpy as jnp
from jax import lax
from jax.experimental import pallas as pl
from jax.experimental.pallas import tpu as pltpu
from jax.experimental.pallas import tpu_sc as plsc

N = 10000
E = 320000
D = 128
OUT = 2
NC = 2
NS = 16
NW = NC * NS
C = 128
NCHUNK = 79
EPAD = NW * NCHUNK * C - E
AROWS = N + 16
RPS = 624
TAIL = N - RPS * NS
ZC = 104

_MESH = plsc.VectorSubcoreMesh(core_axis_name="c", subcore_axis_name="s",
                               num_cores=NC, num_subcores=NS)


def _zero_rows(buf):
    z16 = jnp.zeros((16,), jnp.float32)

    def zrow(i, carry):
        for j in range(D // 16):
            buf[i, pl.ds(j * 16, 16)] = z16
        return carry
    lax.fori_loop(0, C, zrow, 0)


def _zero_my_slice(sh, buf, s):
    row0 = s * RPS
    for k in range(RPS // ZC):
        pltpu.sync_copy(buf.at[pl.ds(0, ZC)],
                        sh.at[pl.ds(row0 + k * ZC, ZC)])

    @pl.when(s == NS - 1)
    def _():
        pltpu.sync_copy(buf.at[pl.ds(0, TAIL)],
                        sh.at[pl.ds(RPS * NS, TAIL)])


def _writeout_my_slice(sh, out, c, s):
    row0 = s * RPS
    pltpu.sync_copy(sh.at[pl.ds(row0, RPS)], out.at[c, pl.ds(row0, RPS)])

    @pl.when(s == NS - 1)
    def _():
        pltpu.sync_copy(sh.at[pl.ds(RPS * NS, TAIL)],
                        out.at[c, pl.ds(RPS * NS, TAIL)])


def _sc_agg_body(feat, srcr, dstr, agg_out, agg_sh, src_v, dst_v, rows_v):
    c = lax.axis_index("c")
    s = lax.axis_index("s")
    wid = c * NS + s

    _zero_rows(rows_v)
    _zero_my_slice(agg_sh, rows_v, s)
    plsc.subcore_barrier()

    def step(j, carry):
        pltpu.sync_copy(srcr.at[wid, j], src_v)
        pltpu.sync_copy(dstr.at[wid, j], dst_v)
        pltpu.sync_copy(feat.at[src_v], rows_v)
        pltpu.sync_copy(rows_v, agg_sh.at[dst_v], add=True)
        return carry
    lax.fori_loop(0, NCHUNK, step, 0)

    plsc.subcore_barrier()
    _writeout_my_slice(agg_sh, agg_out, c, s)


_sc_agg = pl.kernel(
    _sc_agg_body,
    out_type=jax.ShapeDtypeStruct((NC, N, D), jnp.float32),
    mesh=_MESH,
    scratch_types=(
        pltpu.VMEM_SHARED((AROWS, D), jnp.float32),
        pltpu.VMEM((C,), jnp.int32),
        pltpu.VMEM((C,), jnp.int32),
        pltpu.VMEM((C, D), jnp.float32),
    ))


def _sc_deg_body(dstr, deg_out, deg_sh, dst_v, ones_v):
    c = lax.axis_index("c")
    s = lax.axis_index("s")
    wid = c * NS + s

    _zero_rows(ones_v)
    _zero_my_slice(deg_sh, ones_v, s)

    o16 = jnp.full((16,), 1.0, jnp.float32)

    def frow(i, carry):
        for j in range(D // 16):
            ones_v[i, pl.ds(j * 16, 16)] = o16
        return carry
    lax.fori_loop(0, C, frow, 0)

    plsc.subcore_barrier()

    def step(j, carry):
        pltpu.sync_copy(dstr.at[wid, j], dst_v)
        pltpu.sync_copy(ones_v, deg_sh.at[dst_v], add=True)
        return carry
    lax.fori_loop(0, NCHUNK, step, 0)

    plsc.subcore_barrier()
    _writeout_my_slice(deg_sh, deg_out, c, s)


_sc_deg = pl.kernel(
    _sc_deg_body,
    out_type=jax.ShapeDtypeStruct((NC, N, D), jnp.float32),
    mesh=_MESH,
    scratch_types=(
        pltpu.VMEM_SHARED((AROWS, D), jnp.float32),
        pltpu.VMEM((C,), jnp.int32),
        pltpu.VMEM((C, D), jnp.float32),
    ))


_R = 1000


def _tc_layer1(x, aggp, degp, Ws, Wn, b):
    def body(x_r, agg_r, deg_r, ws_r, wn_r, b_r, o_r):
        agg = agg_r[0] + agg_r[1]
        deg = deg_r[0, :, 0:1] + deg_r[1, :, 0:1]
        mean = agg / jnp.maximum(deg, 1.0)
        h = jnp.dot(x_r[...], ws_r[...], preferred_element_type=jnp.float32)
        h = h + jnp.dot(mean, wn_r[...], preferred_element_type=jnp.float32)
        o_r[...] = jnp.maximum(h + b_r[...], 0.0)

    return pl.pallas_call(
        body,
        grid=(N // _R,),
        in_specs=[
            pl.BlockSpec((_R, D), lambda i: (i, 0)),
            pl.BlockSpec((NC, _R, D), lambda i: (0, i, 0)),
            pl.BlockSpec((NC, _R, D), lambda i: (0, i, 0)),
            pl.BlockSpec((D, D), lambda i: (0, 0)),
            pl.BlockSpec((D, D), lambda i: (0, 0)),
            pl.BlockSpec((1, D), lambda i: (0, 0)),
        ],
        out_specs=pl.BlockSpec((_R, D), lambda i: (i, 0)),
        out_shape=jax.ShapeDtypeStruct((N, D), jnp.float32),
    )(x, aggp, degp, Ws, Wn, b.reshape(1, D))


def _tc_layer2_head(h1, aggp, degp, Ws, Wn, b, Wc1, bc1, Wc2p, bc2p):
    def body(h_r, agg_r, deg_r, ws_r, wn_r, b_r, wc1_r, bc1_r, wc2_r,
             bc2_r, o_r):
        agg = agg_r[0] + agg_r[1]
        deg = deg_r[0, :, 0:1] + deg_r[1, :, 0:1]
        mean = agg / jnp.maximum(deg, 1.0)
        h2 = jnp.dot(h_r[...], ws_r[...], preferred_element_type=jnp.float32)
        h2 = h2 + jnp.dot(mean, wn_r[...], preferred_element_type=jnp.float32)
        h2 = h2 + b_r[...]
        z = jnp.maximum(
            jnp.dot(h2, wc1_r[...], preferred_element_type=jnp.float32)
            + bc1_r[...], 0.0)
        o_r[...] = (jnp.dot(z, wc2_r[...], preferred_element_type=jnp.float32)
                    + bc2_r[...])

    full = lambda i: (0, 0)
    return pl.pallas_call(
        body,
        grid=(N // _R,),
        in_specs=[
            pl.BlockSpec((_R, D), lambda i: (i, 0)),
            pl.BlockSpec((NC, _R, D), lambda i: (0, i, 0)),
            pl.BlockSpec((NC, _R, D), lambda i: (0, i, 0)),
            pl.BlockSpec((D, D), full),
            pl.BlockSpec((D, D), full),
            pl.BlockSpec((1, D), full),
            pl.BlockSpec((D, D), full),
            pl.BlockSpec((1, D), full),
            pl.BlockSpec((D, D), full),
            pl.BlockSpec((1, D), full),
        ],
        out_specs=pl.BlockSpec((_R, D), lambda i: (i, 0)),
        out_shape=jax.ShapeDtypeStruct((N, D), jnp.float32),
    )(h1, aggp, degp, Ws, Wn, b.reshape(1, D), Wc1, bc1.reshape(1, D),
      Wc2p, bc2p)


def kernel(x, edge_index, W_self1, W_neigh1, b1, W_self2, W_neigh2, b2,
           Wc1, bc1, Wc2, bc2):
    pad_src = jnp.zeros((EPAD,), jnp.int32)
    pad_dst = jnp.full((EPAD,), N, jnp.int32)
    src = jnp.concatenate([edge_index[0], pad_src]).reshape(NW, NCHUNK, C)
    dst = jnp.concatenate([edge_index[1], pad_dst]).reshape(NW, NCHUNK, C)

    degp = _sc_deg(dst)
    agg1 = _sc_agg(x, src, dst)
    h1 = _tc_layer1(x, agg1, degp, W_self1, W_neigh1, b1)
    agg2 = _sc_agg(h1, src, dst)
    Wc2p = jnp.zeros((D, D), jnp.float32).at[:, :OUT].set(Wc2)
    bc2p = jnp.zeros((1, D), jnp.float32).at[0, :OUT].set(bc2)
    out = _tc_layer2_head(h1, agg2, degp, W_self2, W_neigh2, b2,
                          Wc1, bc1, Wc2p, bc2p)
    return out[:, :OUT]
